# baseline (speedup 1.0000x reference)
import functools

import jax
import jax.numpy as jnp
from jax import lax
from jax.experimental import pallas as pl
from jax.experimental.pallas import tpu as pltpu, tpu_sc as plsc

WORD_COUNT = 1000000
NUM_BUCKETS = 100000
EMBED = 64
BATCH = 4096
SEQ = 200

NC = 2
NS = 16
L = 16
NW = NC * NS

BB = BATCH // NW
META_W = 16


def _sc_body(tok_hbm, tbl_hbm, w_hbm, out_hbm,
             tok_bufs, meta_bufs, idx0_bufs, idx1_bufs, p0_bufs, p1_bufs,
             w0_bufs, w1_bufs, out_bufs,
             tok_sems, meta_sems, w_sems, out_sems):
    wid = lax.axis_index("s") * NC + lax.axis_index("c")
    b0 = wid * BB
    lane = lax.iota(jnp.int32, L)
    zeros = jnp.zeros((L,), jnp.int32)
    ones = jnp.full((L,), 1, jnp.int32)
    twos = jnp.full((L,), 2, jnp.int32)
    threes = jnp.full((L,), 3, jnp.int32)

    def tok_slice(s):
        return tok_hbm.at[s, pl.ds(b0, BB)]

    def out_slice(s):
        return out_hbm.at[:, s, pl.ds(b0, BB)]

    def issue_tok(s):
        pltpu.async_copy(tok_slice(s), tok_bufs.at[s % 4], tok_sems.at[s % 4])

    def wait_tok(s):
        pltpu.make_async_copy(
            tok_slice(s), tok_bufs.at[s % 4], tok_sems.at[s % 4]).wait()

    def issue_meta(s):
        pltpu.async_copy(
            tbl_hbm.at[tok_bufs.at[s % 4]], meta_bufs.at[s % 2],
            meta_sems.at[s % 2])

    def wait_meta(s):
        pltpu.make_async_copy(
            tbl_hbm.at[tok_bufs.at[s % 4]], meta_bufs.at[s % 2],
            meta_sems.at[s % 2]).wait()

    def issue_w(s):
        b = s % 3
        pltpu.async_copy(w_hbm.at[idx0_bufs.at[b]], w0_bufs.at[b],
                         w_sems.at[b])
        pltpu.async_copy(w_hbm.at[idx1_bufs.at[b]], w1_bufs.at[b],
                         w_sems.at[b])

    def wait_w(s):
        b = s % 3
        pltpu.make_async_copy(
            w_hbm.at[idx0_bufs.at[b]], w0_bufs.at[b], w_sems.at[b]).wait()
        pltpu.make_async_copy(
            w_hbm.at[idx1_bufs.at[b]], w1_bufs.at[b], w_sems.at[b]).wait()

    def issue_out(s):
        pltpu.async_copy(out_bufs.at[s % 2], out_slice(s), out_sems.at[s % 2])

    def wait_out(s):
        pltpu.make_async_copy(
            out_bufs.at[s % 2], out_slice(s), out_sems.at[s % 2]).wait()

    def meta_compute(s):
        b = s % 3
        tok_v = tok_bufs.at[s % 4]
        meta_v = meta_bufs.at[s % 2]
        idx0_v = idx0_bufs.at[b]
        idx1_v = idx1_bufs.at[b]
        p0_v = p0_bufs.at[b]
        p1_v = p1_bufs.at[b]

        def grp(g, _):
            gs = g * L
            rows = gs + lane
            wv = tok_v[pl.ds(gs, L)]
            nz = wv != 0
            bk0 = plsc.load_gather(meta_v, [rows, zeros])
            bk1 = plsc.load_gather(meta_v, [rows, ones])
            p0b = plsc.load_gather(meta_v, [rows, twos])
            p1b = plsc.load_gather(meta_v, [rows, threes])
            idx0_v[pl.ds(gs, L)] = jnp.where(nz, bk0, 0)
            idx1_v[pl.ds(gs, L)] = jnp.where(nz, bk1, 0)
            p0_v[pl.ds(gs, L)] = plsc.bitcast(p0b, jnp.float32)
            p1_v[pl.ds(gs, L)] = plsc.bitcast(p1b, jnp.float32)
            return 0

        lax.fori_loop(0, BB // L, grp, 0)

    def out_compute(s):
        b = s % 3
        p0_v = p0_bufs.at[b]
        p1_v = p1_bufs.at[b]
        w0_v = w0_bufs.at[b]
        w1_v = w1_bufs.at[b]
        out_v = out_bufs.at[s % 2]

        def feat(e, _):
            ev = jnp.full((L,), e, jnp.int32)
            for g in range(BB // L):
                gs = g * L
                rows = gs + lane
                a = plsc.load_gather(w0_v, [rows, ev])
                c = plsc.load_gather(w1_v, [rows, ev])
                p0 = p0_v[pl.ds(gs, L)]
                p1 = p1_v[pl.ds(gs, L)]
                out_v[e, pl.ds(gs, L)] = a * p0 + c * p1
            return 0

        lax.fori_loop(0, EMBED, feat, 0)

        def tails(g, _):
            gs = g * L
            out_v[EMBED, pl.ds(gs, L)] = p0_v[pl.ds(gs, L)]
            out_v[EMBED + 1, pl.ds(gs, L)] = p1_v[pl.ds(gs, L)]
            return 0

        lax.fori_loop(0, BB // L, tails, 0)

    pltpu.sync_copy(tok_slice(0), tok_bufs.at[0])
    pltpu.sync_copy(tok_slice(1), tok_bufs.at[1])
    issue_tok(2)
    issue_tok(3)
    issue_meta(0)
    issue_meta(1)
    wait_meta(0)
    meta_compute(0)
    issue_w(0)
    wait_tok(2)
    issue_meta(2)
    wait_meta(1)
    meta_compute(1)
    issue_w(1)

    def iter_body(s, carry):
        wait_tok(s + 3)
        issue_meta(s + 3)

        @pl.when(s + 4 <= SEQ - 1)
        def _():
            issue_tok(s + 4)

        wait_meta(s + 2)
        meta_compute(s + 2)
        issue_w(s + 2)

        @pl.when(s >= 2)
        def _():
            wait_out(s - 2)

        wait_w(s)
        out_compute(s)
        issue_out(s)
        return carry

    lax.fori_loop(0, SEQ - 3, iter_body, 0)

    wait_meta(SEQ - 1)
    meta_compute(SEQ - 1)
    issue_w(SEQ - 1)
    for s in (SEQ - 3, SEQ - 2, SEQ - 1):
        wait_out(s - 2)
        wait_w(s)
        out_compute(s)
        issue_out(s)
    wait_out(SEQ - 2)
    wait_out(SEQ - 1)


def kernel(input, hash_tables, p, W):
    tok_t = input.T
    p_shift = jnp.roll(p, -3, axis=0)
    tbl = jnp.concatenate(
        [hash_tables,
         lax.bitcast_convert_type(p_shift, jnp.int32),
         jnp.zeros((WORD_COUNT, META_W - 4), jnp.int32)], axis=1)

    mesh = plsc.VectorSubcoreMesh(
        core_axis_name="c", subcore_axis_name="s",
        num_cores=NC, num_subcores=NS)
    run = pl.kernel(
        _sc_body,
        out_type=jax.ShapeDtypeStruct((EMBED + 2, SEQ, BATCH), jnp.float32),
        mesh=mesh,
        compiler_params=pltpu.CompilerParams(
            needs_layout_passes=False, use_tc_tiling_on_sc=False),
        scratch_types=[
            pltpu.VMEM((4, BB), jnp.int32),
            pltpu.VMEM((2, BB, META_W), jnp.int32),
            pltpu.VMEM((3, BB), jnp.int32),
            pltpu.VMEM((3, BB), jnp.int32),
            pltpu.VMEM((3, BB), jnp.float32),
            pltpu.VMEM((3, BB), jnp.float32),
            pltpu.VMEM((3, BB, EMBED), jnp.float32),
            pltpu.VMEM((3, BB, EMBED), jnp.float32),
            pltpu.VMEM((2, EMBED + 2, BB), jnp.float32),
            pltpu.SemaphoreType.DMA((4,)),
            pltpu.SemaphoreType.DMA((2,)),
            pltpu.SemaphoreType.DMA((3,)),
            pltpu.SemaphoreType.DMA((2,)),
        ],
    )
    out = run(tok_t, tbl, W)
    return out.transpose(2, 1, 0)


# speedup vs baseline: 2.3003x; 2.3003x over previous
import functools

import jax
import jax.numpy as jnp
from jax import lax
from jax.experimental import pallas as pl
from jax.experimental.pallas import tpu as pltpu, tpu_sc as plsc

WORD_COUNT = 1000000
NUM_BUCKETS = 100000
EMBED = 64
BATCH = 4096
SEQ = 200

NC = 2
NS = 16
L = 16
NW = NC * NS

BB = BATCH // NW
META_W = 16


def _sc_body(tok_hbm, tbl_hbm, w_hbm, out_hbm,
             tok_bufs, meta_bufs, idx0_bufs, idx1_bufs, p0_bufs, p1_bufs,
             w0_bufs, w1_bufs, out_bufs,
             tok_sems, meta_sems, w_sems, out_sems):
    wid = lax.axis_index("s") * NC + lax.axis_index("c")
    b0 = wid * BB
    lane = lax.iota(jnp.int32, L)
    zeros = jnp.zeros((L,), jnp.int32)
    ones = jnp.full((L,), 1, jnp.int32)
    twos = jnp.full((L,), 2, jnp.int32)
    threes = jnp.full((L,), 3, jnp.int32)

    def tok_slice(s):
        return tok_hbm.at[s, pl.ds(b0, BB)]

    def out_slice(s):
        return out_hbm.at[:, s, pl.ds(b0, BB)]

    def issue_tok(s):
        pltpu.async_copy(tok_slice(s), tok_bufs.at[s % 4], tok_sems.at[s % 4])

    def wait_tok(s):
        pltpu.make_async_copy(
            tok_slice(s), tok_bufs.at[s % 4], tok_sems.at[s % 4]).wait()

    def issue_meta(s):
        pltpu.async_copy(
            tbl_hbm.at[tok_bufs.at[s % 4]], meta_bufs.at[s % 2],
            meta_sems.at[s % 2])

    def wait_meta(s):
        pltpu.make_async_copy(
            tbl_hbm.at[tok_bufs.at[s % 4]], meta_bufs.at[s % 2],
            meta_sems.at[s % 2]).wait()

    def issue_w(s):
        b = s % 3
        pltpu.async_copy(w_hbm.at[idx0_bufs.at[b]], w0_bufs.at[b],
                         w_sems.at[b])
        pltpu.async_copy(w_hbm.at[idx1_bufs.at[b]], w1_bufs.at[b],
                         w_sems.at[b])

    def wait_w(s):
        b = s % 3
        pltpu.make_async_copy(
            w_hbm.at[idx0_bufs.at[b]], w0_bufs.at[b], w_sems.at[b]).wait()
        pltpu.make_async_copy(
            w_hbm.at[idx1_bufs.at[b]], w1_bufs.at[b], w_sems.at[b]).wait()

    def issue_out(s):
        pltpu.async_copy(out_bufs.at[s % 2], out_slice(s), out_sems.at[s % 2])

    def wait_out(s):
        pltpu.make_async_copy(
            out_bufs.at[s % 2], out_slice(s), out_sems.at[s % 2]).wait()

    def meta_compute(s):
        b = s % 3
        tok_v = tok_bufs.at[s % 4]
        meta_v = meta_bufs.at[s % 2]
        idx0_v = idx0_bufs.at[b]
        idx1_v = idx1_bufs.at[b]
        p0_v = p0_bufs.at[b]
        p1_v = p1_bufs.at[b]

        def grp(g, _):
            gs = g * L
            rows = gs + lane
            wv = tok_v[pl.ds(gs, L)]
            nz = wv != 0
            bk0 = plsc.load_gather(meta_v, [rows, zeros])
            bk1 = plsc.load_gather(meta_v, [rows, ones])
            p0b = plsc.load_gather(meta_v, [rows, twos])
            p1b = plsc.load_gather(meta_v, [rows, threes])
            idx0_v[pl.ds(gs, L)] = jnp.where(nz, bk0, 0)
            idx1_v[pl.ds(gs, L)] = jnp.where(nz, bk1, 0)
            p0_v[pl.ds(gs, L)] = plsc.bitcast(p0b, jnp.float32)
            p1_v[pl.ds(gs, L)] = plsc.bitcast(p1b, jnp.float32)
            return 0

        lax.fori_loop(0, BB // L, grp, 0)

    perms = [(lane + d) & (L - 1) for d in range(L)]

    def out_compute(s):
        b = s % 3
        p0_v = p0_bufs.at[b]
        p1_v = p1_bufs.at[b]
        w0_v = w0_bufs.at[b]
        w1_v = w1_bufs.at[b]
        out_v = out_bufs.at[s % 2]

        def blk(i, _):
            g = i // (EMBED // L)
            k = i - g * (EMBED // L)
            gs = g * L
            e0 = k * L
            rows = gs + lane
            p0 = p0_v[pl.ds(gs, L)]
            p1 = p1_v[pl.ds(gs, L)]
            for d in range(L):
                ecol = e0 + perms[d]
                a = plsc.load_gather(w0_v, [rows, ecol])
                c = plsc.load_gather(w1_v, [rows, ecol])
                plsc.store_scatter(out_v, [ecol, rows], a * p0 + c * p1)
            return 0

        lax.fori_loop(0, (BB // L) * (EMBED // L), blk, 0)

        def tails(g, _):
            gs = g * L
            out_v[EMBED, pl.ds(gs, L)] = p0_v[pl.ds(gs, L)]
            out_v[EMBED + 1, pl.ds(gs, L)] = p1_v[pl.ds(gs, L)]
            return 0

        lax.fori_loop(0, BB // L, tails, 0)

    pltpu.sync_copy(tok_slice(0), tok_bufs.at[0])
    pltpu.sync_copy(tok_slice(1), tok_bufs.at[1])
    issue_tok(2)
    issue_tok(3)
    issue_meta(0)
    issue_meta(1)
    wait_meta(0)
    meta_compute(0)
    issue_w(0)
    wait_tok(2)
    issue_meta(2)
    wait_meta(1)
    meta_compute(1)
    issue_w(1)

    def iter_body(s, carry):
        wait_tok(s + 3)
        issue_meta(s + 3)

        @pl.when(s + 4 <= SEQ - 1)
        def _():
            issue_tok(s + 4)

        wait_meta(s + 2)
        meta_compute(s + 2)
        issue_w(s + 2)

        @pl.when(s >= 2)
        def _():
            wait_out(s - 2)

        wait_w(s)
        out_compute(s)
        issue_out(s)
        return carry

    lax.fori_loop(0, SEQ - 3, iter_body, 0)

    wait_meta(SEQ - 1)
    meta_compute(SEQ - 1)
    issue_w(SEQ - 1)
    for s in (SEQ - 3, SEQ - 2, SEQ - 1):
        wait_out(s - 2)
        wait_w(s)
        out_compute(s)
        issue_out(s)
    wait_out(SEQ - 2)
    wait_out(SEQ - 1)


def kernel(input, hash_tables, p, W):
    tok_t = input.T
    p_shift = jnp.roll(p, -3, axis=0)
    tbl = jnp.concatenate(
        [hash_tables,
         lax.bitcast_convert_type(p_shift, jnp.int32),
         jnp.zeros((WORD_COUNT, META_W - 4), jnp.int32)], axis=1)

    mesh = plsc.VectorSubcoreMesh(
        core_axis_name="c", subcore_axis_name="s",
        num_cores=NC, num_subcores=NS)
    run = pl.kernel(
        _sc_body,
        out_type=jax.ShapeDtypeStruct((EMBED + 2, SEQ, BATCH), jnp.float32),
        mesh=mesh,
        compiler_params=pltpu.CompilerParams(
            needs_layout_passes=False, use_tc_tiling_on_sc=False),
        scratch_types=[
            pltpu.VMEM((4, BB), jnp.int32),
            pltpu.VMEM((2, BB, META_W), jnp.int32),
            pltpu.VMEM((3, BB), jnp.int32),
            pltpu.VMEM((3, BB), jnp.int32),
            pltpu.VMEM((3, BB), jnp.float32),
            pltpu.VMEM((3, BB), jnp.float32),
            pltpu.VMEM((3, BB, EMBED), jnp.float32),
            pltpu.VMEM((3, BB, EMBED), jnp.float32),
            pltpu.VMEM((2, EMBED + 2, BB), jnp.float32),
            pltpu.SemaphoreType.DMA((4,)),
            pltpu.SemaphoreType.DMA((2,)),
            pltpu.SemaphoreType.DMA((3,)),
            pltpu.SemaphoreType.DMA((2,)),
        ],
    )
    out = run(tok_t, tbl, W)
    return out.transpose(2, 1, 0)
